# baseline (speedup 1.0000x reference)
import jax
import jax.numpy as jnp
from jax import lax
from jax.experimental import pallas as pl
from jax.experimental.pallas import tpu as pltpu
from jax.experimental.pallas import tpu_sc as plsc

D_MODEL = 1024
SEQ_LEN = 2048
BATCH = 4
SCALE = 32.0

NC, NS, L = 2, 16, 16
NW = NC * NS
NROWS = SEQ_LEN * BATCH
ROWS_PER_W = NROWS // NW
CHUNK = 8
SEQ_PER_CHUNK = CHUNK // BATCH
NCHUNK = ROWS_PER_W // CHUNK
SEQ_PER_W = ROWS_PER_W // BATCH
PE_HALF = SEQ_PER_W // 2
NGBUF = 4
NOBUF = 2
KITER = NCHUNK // NGBUF


def _body(idx_hbm, pe_hbm, table_hbm, out_hbm, idx_v, pe_v,
          g0, g1, g2, g3, o0, o1, gs0, gs1, gs2, gs3, os0, os1, psem):
    wid = lax.axis_index("s") * NC + lax.axis_index("c")
    base = wid * ROWS_PER_W
    seq_base = wid * SEQ_PER_W
    pltpu.sync_copy(idx_hbm.at[pl.ds(base, ROWS_PER_W)], idx_v)
    pe_copy = pltpu.async_copy(
        pe_hbm.at[pl.ds(seq_base, SEQ_PER_W)], pe_v, psem)

    gbufs = (g0, g1, g2, g3)
    obufs = (o0, o1)
    gsems = (gs0, gs1, gs2, gs3)
    osems = (os0, os1)

    def start_gather(c, i):
        pltpu.async_copy(
            table_hbm.at[idx_v.at[pl.ds(c * CHUNK, CHUNK)]], gbufs[i], gsems[i])

    def start_out(c, j):
        pltpu.async_copy(
            obufs[j],
            out_hbm.at[pl.ds(seq_base + c * SEQ_PER_CHUNK, SEQ_PER_CHUNK)],
            osems[j])

    def drain_gather(i):
        pltpu.make_async_copy(
            table_hbm.at[pl.ds(0, CHUNK)], gbufs[i], gsems[i]).wait()

    def drain_out(j):
        pltpu.make_async_copy(
            out_hbm.at[pl.ds(0, SEQ_PER_CHUNK)], obufs[j], osems[j]).wait()

    def compute(c, i, j):
        gbuf = gbufs[i]
        obuf = obufs[j]

        @plsc.parallel_loop(0, D_MODEL, step=L, unroll=16)
        def _(col0):
            col = pl.ds(pl.multiple_of(col0, L), L)
            for g in range(SEQ_PER_CHUNK):
                pv = pe_v[c * SEQ_PER_CHUNK + g, 0, col]
                for r in range(BATCH):
                    obuf[g, r, col] = gbuf[g * BATCH + r, col] * SCALE + pv

    for i in range(NGBUF):
        start_gather(i, i)
    pe_copy.wait()

    def kbody(k, carry):
        for i in range(NGBUF):
            c = NGBUF * k + i
            j = i % NOBUF
            drain_gather(i)

            if i < NOBUF:
                @pl.when(k > 0)
                def _():
                    drain_out(j)
            else:
                drain_out(j)

            compute(c, i, j)
            start_out(c, j)

            @pl.when(k < KITER - 1)
            def _():
                start_gather(c + NGBUF, i)
        return carry

    lax.fori_loop(0, KITER, kbody, 0)
    for j in range(NOBUF):
        drain_out(j)


_mesh = plsc.VectorSubcoreMesh(core_axis_name="c", subcore_axis_name="s")

_emb = pl.kernel(
    _body,
    mesh=_mesh,
    out_type=jax.ShapeDtypeStruct((SEQ_LEN, BATCH, D_MODEL), jnp.float32),
    scratch_types=[
        pltpu.VMEM((ROWS_PER_W,), jnp.int32),
        pltpu.VMEM((SEQ_PER_W, 1, D_MODEL), jnp.float32),
        pltpu.VMEM((CHUNK, D_MODEL), jnp.float32),
        pltpu.VMEM((CHUNK, D_MODEL), jnp.float32),
        pltpu.VMEM((CHUNK, D_MODEL), jnp.float32),
        pltpu.VMEM((CHUNK, D_MODEL), jnp.float32),
        pltpu.VMEM((SEQ_PER_CHUNK, BATCH, D_MODEL), jnp.float32),
        pltpu.VMEM((SEQ_PER_CHUNK, BATCH, D_MODEL), jnp.float32),
        pltpu.SemaphoreType.DMA,
        pltpu.SemaphoreType.DMA,
        pltpu.SemaphoreType.DMA,
        pltpu.SemaphoreType.DMA,
        pltpu.SemaphoreType.DMA,
        pltpu.SemaphoreType.DMA,
        pltpu.SemaphoreType.DMA,
    ],
)


def kernel(x, table, pe):
    idx = x.reshape(-1).astype(jnp.int32)
    return _emb(idx, pe, table)


# speedup vs baseline: 1.0167x; 1.0167x over previous
import jax
import jax.numpy as jnp
from jax import lax
from jax.experimental import pallas as pl
from jax.experimental.pallas import tpu as pltpu
from jax.experimental.pallas import tpu_sc as plsc

D_MODEL = 1024
SEQ_LEN = 2048
BATCH = 4
SCALE = 32.0

NC, NS, L = 2, 16, 16
NW = NC * NS
NROWS = SEQ_LEN * BATCH
ROWS_PER_W = NROWS // NW
CHUNK = 8
SEQ_PER_CHUNK = CHUNK // BATCH
NCHUNK = ROWS_PER_W // CHUNK
SEQ_PER_W = ROWS_PER_W // BATCH
PE_QUARTER = SEQ_PER_W // 4
NGBUF = 4
NOBUF = 2
KITER = NCHUNK // NGBUF


def _body(idx_hbm, pe_hbm, table_hbm, out_hbm, idx_v, pe_v,
          g0, g1, g2, g3, o0, o1, gs0, gs1, gs2, gs3, os0, os1, psem):
    wid = lax.axis_index("s") * NC + lax.axis_index("c")
    base = wid * ROWS_PER_W
    seq_base = wid * SEQ_PER_W
    pltpu.sync_copy(idx_hbm.at[pl.ds(base, ROWS_PER_W)], idx_v)
    pltpu.async_copy(
        pe_hbm.at[pl.ds(seq_base, PE_QUARTER)], pe_v.at[pl.ds(0, PE_QUARTER)],
        psem)

    gbufs = (g0, g1, g2, g3)
    obufs = (o0, o1)
    gsems = (gs0, gs1, gs2, gs3)
    osems = (os0, os1)

    def start_gather(c, i):
        pltpu.async_copy(
            table_hbm.at[idx_v.at[pl.ds(c * CHUNK, CHUNK)]], gbufs[i], gsems[i])

    def start_out(c, j):
        pltpu.async_copy(
            obufs[j],
            out_hbm.at[pl.ds(seq_base + c * SEQ_PER_CHUNK, SEQ_PER_CHUNK)],
            osems[j])

    def drain_gather(i):
        pltpu.make_async_copy(
            table_hbm.at[pl.ds(0, CHUNK)], gbufs[i], gsems[i]).wait()

    def drain_out(j):
        pltpu.make_async_copy(
            out_hbm.at[pl.ds(0, SEQ_PER_CHUNK)], obufs[j], osems[j]).wait()

    def compute(c, i, j):
        gbuf = gbufs[i]
        obuf = obufs[j]

        @plsc.parallel_loop(0, D_MODEL, step=L, unroll=16)
        def _(col0):
            col = pl.ds(pl.multiple_of(col0, L), L)
            for g in range(SEQ_PER_CHUNK):
                pv = pe_v[c * SEQ_PER_CHUNK + g, 0, col]
                for r in range(BATCH):
                    obuf[g, r, col] = gbuf[g * BATCH + r, col] * SCALE + pv

    def drain_pe():
        pltpu.make_async_copy(
            pe_hbm.at[pl.ds(0, PE_QUARTER)],
            pe_v.at[pl.ds(0, PE_QUARTER)], psem).wait()

    for i in range(NGBUF):
        start_gather(i, i)
    for q in range(1, 4):
        pltpu.async_copy(
            pe_hbm.at[pl.ds(seq_base + q * PE_QUARTER, PE_QUARTER)],
            pe_v.at[pl.ds(q * PE_QUARTER, PE_QUARTER)], psem)
    drain_pe()

    def kbody(k, carry):
        @pl.when(jnp.logical_and(k > 0, lax.rem(k, 2) == 0))
        def _():
            drain_pe()

        for i in range(NGBUF):
            c = NGBUF * k + i
            j = i % NOBUF
            drain_gather(i)

            if i < NOBUF:
                @pl.when(k > 0)
                def _():
                    drain_out(j)
            else:
                drain_out(j)

            compute(c, i, j)
            start_out(c, j)

            @pl.when(k < KITER - 1)
            def _():
                start_gather(c + NGBUF, i)
        return carry

    lax.fori_loop(0, KITER, kbody, 0)
    for j in range(NOBUF):
        drain_out(j)


_mesh = plsc.VectorSubcoreMesh(core_axis_name="c", subcore_axis_name="s")

_emb = pl.kernel(
    _body,
    mesh=_mesh,
    out_type=jax.ShapeDtypeStruct((SEQ_LEN, BATCH, D_MODEL), jnp.float32),
    scratch_types=[
        pltpu.VMEM((ROWS_PER_W,), jnp.int32),
        pltpu.VMEM((SEQ_PER_W, 1, D_MODEL), jnp.float32),
        pltpu.VMEM((CHUNK, D_MODEL), jnp.float32),
        pltpu.VMEM((CHUNK, D_MODEL), jnp.float32),
        pltpu.VMEM((CHUNK, D_MODEL), jnp.float32),
        pltpu.VMEM((CHUNK, D_MODEL), jnp.float32),
        pltpu.VMEM((SEQ_PER_CHUNK, BATCH, D_MODEL), jnp.float32),
        pltpu.VMEM((SEQ_PER_CHUNK, BATCH, D_MODEL), jnp.float32),
        pltpu.SemaphoreType.DMA,
        pltpu.SemaphoreType.DMA,
        pltpu.SemaphoreType.DMA,
        pltpu.SemaphoreType.DMA,
        pltpu.SemaphoreType.DMA,
        pltpu.SemaphoreType.DMA,
        pltpu.SemaphoreType.DMA,
    ],
)


def kernel(x, table, pe):
    idx = x.reshape(-1).astype(jnp.int32)
    return _emb(idx, pe, table)
